# Initial kernel scaffold; baseline (speedup 1.0000x reference)
import jax
import jax.numpy as jnp
from jax.experimental import pallas as pl


def kernel(x, W_gate):
    raise NotImplementedError("write your pallas kernel here")


# speedup vs baseline: 1.3810x; 1.3810x over previous
import functools
import math

import jax
import jax.numpy as jnp
from jax import lax
from jax.experimental import pallas as pl
from jax.experimental.pallas import tpu as pltpu
from jax.experimental.pallas import tpu_sc as plsc

D_MODEL = 1024
N_EXPERTS = 8
TOP_K = 2
N_TOKENS = 2048
CAP = math.floor(TOP_K * 1.25 * N_TOKENS / N_EXPERTS)
CAP += CAP % 2
SLOTS = N_EXPERTS * CAP
DUMMY_SLOT = SLOTS
PAD_ROW = N_TOKENS

LANES = 128
BLK_A = 256
BLK_B = 256


def _meta_body(x_ref, wt_ref, kept_ref, wk_ref, slots_ref, smask_ref, carry_ref):
    i = pl.program_id(0)

    @pl.when(i == 0)
    def _():
        carry_ref[...] = jnp.zeros_like(carry_ref)

    g = jnp.dot(x_ref[...], wt_ref[...], preferred_element_type=jnp.float32)
    lane = lax.broadcasted_iota(jnp.int32, (BLK_A, LANES), 1)
    neg = jnp.float32(-1e30)
    g = jnp.where(lane < N_EXPERTS, g, neg)

    m1 = jnp.max(g, axis=1, keepdims=True)
    i1 = jnp.min(jnp.where(g == m1, lane, LANES - 1), axis=1, keepdims=True)
    g2 = jnp.where(lane == i1, neg, g)
    m2 = jnp.max(g2, axis=1, keepdims=True)
    i2 = jnp.min(jnp.where(g2 == m2, lane, LANES - 1), axis=1, keepdims=True)

    t = jnp.exp(m2 - m1)
    denom = 1.0 + t
    ew1 = 1.0 / denom
    ew2 = t / denom
    sel1 = lane == i1
    sel2 = lane == i2
    self32 = jnp.where(sel1 | sel2, 1.0, 0.0).astype(jnp.float32)
    wte = jnp.where(sel1, ew1, 0.0) + jnp.where(sel2, ew2, 0.0)

    r_ = lax.broadcasted_iota(jnp.int32, (BLK_A, BLK_A), 0)
    c_ = lax.broadcasted_iota(jnp.int32, (BLK_A, BLK_A), 1)
    tril = jnp.where(r_ >= c_, 1.0, 0.0).astype(jnp.float32)
    cs = jnp.dot(tril, self32, preferred_element_type=jnp.float32)

    carry = carry_ref[0:1, :]
    rank = carry + cs - 1.0
    keptf = self32 * jnp.where(rank < CAP, 1.0, 0.0)
    kept_ref[...] = keptf
    wk_ref[...] = keptf * wte

    carry_new = carry + cs[BLK_A - 1:BLK_A, :]
    carry_ref[0:1, :] = carry_new

    rank1 = jnp.sum(jnp.where(sel1, rank, 0.0), axis=1, keepdims=True)
    rank2 = jnp.sum(jnp.where(sel2, rank, 0.0), axis=1, keepdims=True)
    s1 = jnp.where(rank1 < CAP, i1 * CAP + rank1.astype(jnp.int32), DUMMY_SLOT)
    s2 = jnp.where(rank2 < CAP, i2 * CAP + rank2.astype(jnp.int32), DUMMY_SLOT)
    slots_ref[...] = jnp.where(lane == 0, s1, jnp.where(lane == 1, s2, DUMMY_SLOT))

    lf = jnp.minimum(carry_new, jnp.float32(CAP))
    sub8 = lax.broadcasted_iota(jnp.int32, (N_EXPERTS, LANES), 0)
    lane8 = lax.broadcasted_iota(jnp.int32, (N_EXPERTS, LANES), 1)
    lcol = jnp.sum(
        jnp.where(sub8 == lane8, jnp.broadcast_to(lf, (N_EXPERTS, LANES)), 0.0),
        axis=1, keepdims=True)
    cpos = lax.broadcasted_iota(jnp.int32, (N_EXPERTS, CAP), 1).astype(jnp.float32)
    smask_ref[...] = jnp.where(cpos < lcol, 1.0, 0.0)


def _make_meta_call(interpret=False):
    return pl.pallas_call(
        _meta_body,
        interpret=interpret,
        grid=(N_TOKENS // BLK_A,),
        in_specs=[
            pl.BlockSpec((BLK_A, D_MODEL), lambda i: (i, 0)),
            pl.BlockSpec((D_MODEL, LANES), lambda i: (0, 0)),
        ],
        out_specs=[
            pl.BlockSpec((BLK_A, LANES), lambda i: (i, 0)),
            pl.BlockSpec((BLK_A, LANES), lambda i: (i, 0)),
            pl.BlockSpec((BLK_A, LANES), lambda i: (i, 0)),
            pl.BlockSpec((N_EXPERTS, CAP), lambda i: (0, 0)),
        ],
        out_shape=[
            jax.ShapeDtypeStruct((N_TOKENS, LANES), jnp.float32),
            jax.ShapeDtypeStruct((N_TOKENS, LANES), jnp.float32),
            jax.ShapeDtypeStruct((N_TOKENS, LANES), jnp.int32),
            jax.ShapeDtypeStruct((N_EXPERTS, CAP), jnp.float32),
        ],
        scratch_shapes=[pltpu.VMEM((8, LANES), jnp.float32)],
    )


_meta_call = _make_meta_call()


def _write_body(kept_ref, wk_ref, smask_ref, fw_ref, mk_ref):
    k8 = kept_ref[:, 0:N_EXPERTS]
    w8 = wk_ref[:, 0:N_EXPERTS]
    sm = smask_ref[...]
    mk_ref[...] = k8[:, :, None] * sm[None, :, :]
    fw_ref[...] = w8[:, :, None] * sm[None, :, :]


def _make_write_call(interpret=False):
    return pl.pallas_call(
        _write_body,
        interpret=interpret,
        grid=(N_TOKENS // BLK_B,),
        in_specs=[
            pl.BlockSpec((BLK_B, LANES), lambda i: (i, 0)),
            pl.BlockSpec((BLK_B, LANES), lambda i: (i, 0)),
            pl.BlockSpec((N_EXPERTS, CAP), lambda i: (0, 0)),
        ],
        out_specs=[
            pl.BlockSpec((BLK_B, N_EXPERTS, CAP), lambda i: (i, 0, 0)),
            pl.BlockSpec((BLK_B, N_EXPERTS, CAP), lambda i: (i, 0, 0)),
        ],
        out_shape=[
            jax.ShapeDtypeStruct((N_TOKENS, N_EXPERTS, CAP), jnp.float32),
            jax.ShapeDtypeStruct((N_TOKENS, N_EXPERTS, CAP), jnp.float32),
        ],
    )


_write_call = _make_write_call()


_SC_NC = 2
_SC_NS = 16
_SC_NW = _SC_NC * _SC_NS
_TBL = 5152
_PER_W = SLOTS // _SC_NW
_CHUNK = 80


def _gather_body(slots_hbm, xpad_hbm, out_hbm, tbl_v, slots_v, tok_v, tbl_sh,
                 myidx_v, rows_v, sem):
    cid = lax.axis_index("c")
    sid = lax.axis_index("s")
    wid = sid * _SC_NC + cid

    @pl.when(sid == 0)
    def _build():
        def _init(i, c):
            tbl_v[pl.ds(i * 16, 16)] = jnp.full((16,), PAD_ROW, jnp.int32)
            return c
        lax.fori_loop(0, _TBL // 16, _init, 0)
        pltpu.sync_copy(tbl_v, tbl_sh)
        pltpu.sync_copy(slots_hbm, slots_v)

        def _tok(i, c):
            tok_v[pl.ds(i * 16, 16)] = (i * 16 + lax.iota(jnp.int32, 16)) >> 1
            return c
        lax.fori_loop(0, (N_TOKENS * TOP_K) // 16, _tok, 0)
        pltpu.sync_copy(tok_v, tbl_sh.at[slots_v])

    plsc.subcore_barrier()
    base = wid * _PER_W
    pltpu.sync_copy(tbl_sh.at[pl.ds(base, _PER_W)], myidx_v)
    for ch in range(_PER_W // _CHUNK):
        cbase = ch * _CHUNK
        pltpu.async_copy(
            xpad_hbm.at[myidx_v.at[pl.ds(cbase, _CHUNK)]], rows_v, sem).wait()
        pltpu.sync_copy(rows_v, out_hbm.at[pl.ds(base + cbase, _CHUNK)])


@functools.cache
def _gather_call():
    return functools.partial(
        pl.kernel,
        mesh=plsc.VectorSubcoreMesh(core_axis_name="c", subcore_axis_name="s"),
        out_type=jax.ShapeDtypeStruct((SLOTS, D_MODEL), jnp.float32),
        scratch_types=[
            pltpu.VMEM((_TBL,), jnp.int32),
            pltpu.VMEM((N_TOKENS * TOP_K,), jnp.int32),
            pltpu.VMEM((N_TOKENS * TOP_K,), jnp.int32),
            pltpu.VMEM_SHARED((_TBL,), jnp.int32),
            pltpu.VMEM((_PER_W,), jnp.int32),
            pltpu.VMEM((_CHUNK, D_MODEL), jnp.float32),
            pltpu.SemaphoreType.DMA,
        ],
    )(_gather_body)


def kernel(x, W_gate):
    xf = x.reshape(N_TOKENS, D_MODEL)
    wt = jnp.zeros((D_MODEL, LANES), jnp.float32).at[:, :N_EXPERTS].set(W_gate.T)
    keptf, wk, slots128, smask = _meta_call(xf, wt)
    slots = slots128[:, :TOP_K].reshape(N_TOKENS * TOP_K)
    fw, mk = _write_call(keptf, wk, smask)
    xpad = jnp.zeros((N_TOKENS + 1, D_MODEL), jnp.float32).at[:N_TOKENS].set(xf)
    eb = _gather_call()(slots, xpad)
    return fw, mk, eb.reshape(N_EXPERTS, CAP, D_MODEL)


# speedup vs baseline: 1.4372x; 1.0407x over previous
import functools
import math

import jax
import jax.numpy as jnp
from jax import lax
from jax.experimental import pallas as pl
from jax.experimental.pallas import tpu as pltpu
from jax.experimental.pallas import tpu_sc as plsc

D_MODEL = 1024
N_EXPERTS = 8
TOP_K = 2
N_TOKENS = 2048
CAP = math.floor(TOP_K * 1.25 * N_TOKENS / N_EXPERTS)
CAP += CAP % 2
SLOTS = N_EXPERTS * CAP
DUMMY_SLOT = SLOTS
PAD_ROW = N_TOKENS

LANES = 128
BLK_A = 256
BLK_B = 256


def _meta_body(x_ref, wt_ref, kept_ref, wk_ref, slots_ref, smask_ref,
               xpad_ref, carry_ref):
    i = pl.program_id(0)

    @pl.when(i == 0)
    def _():
        carry_ref[...] = jnp.zeros_like(carry_ref)

    @pl.when(i >= N_TOKENS // BLK_A)
    def _():
        xpad_ref[...] = jnp.zeros_like(xpad_ref)

    @pl.when(i < N_TOKENS // BLK_A)
    def _main():
        _meta_step(x_ref, wt_ref, kept_ref, wk_ref, slots_ref, smask_ref,
                   xpad_ref, carry_ref)


def _meta_step(x_ref, wt_ref, kept_ref, wk_ref, slots_ref, smask_ref,
               xpad_ref, carry_ref):
    xpad_ref[...] = x_ref[...]
    g = jnp.dot(x_ref[...], wt_ref[...], preferred_element_type=jnp.float32)
    lane = lax.broadcasted_iota(jnp.int32, (BLK_A, LANES), 1)
    neg = jnp.float32(-1e30)
    g = jnp.where(lane < N_EXPERTS, g, neg)

    m1 = jnp.max(g, axis=1, keepdims=True)
    i1 = jnp.min(jnp.where(g == m1, lane, LANES - 1), axis=1, keepdims=True)
    g2 = jnp.where(lane == i1, neg, g)
    m2 = jnp.max(g2, axis=1, keepdims=True)
    i2 = jnp.min(jnp.where(g2 == m2, lane, LANES - 1), axis=1, keepdims=True)

    t = jnp.exp(m2 - m1)
    denom = 1.0 + t
    ew1 = 1.0 / denom
    ew2 = t / denom
    sel1 = lane == i1
    sel2 = lane == i2
    self32 = jnp.where(sel1 | sel2, 1.0, 0.0).astype(jnp.float32)
    wte = jnp.where(sel1, ew1, 0.0) + jnp.where(sel2, ew2, 0.0)

    r_ = lax.broadcasted_iota(jnp.int32, (BLK_A, BLK_A), 0)
    c_ = lax.broadcasted_iota(jnp.int32, (BLK_A, BLK_A), 1)
    tril = jnp.where(r_ >= c_, 1.0, 0.0).astype(jnp.float32)
    cs = jnp.dot(tril, self32, preferred_element_type=jnp.float32)

    carry = carry_ref[0:1, :]
    rank = carry + cs - 1.0
    keptf = self32 * jnp.where(rank < CAP, 1.0, 0.0)
    kept_ref[...] = keptf
    wk_ref[...] = keptf * wte

    carry_new = carry + cs[BLK_A - 1:BLK_A, :]
    carry_ref[0:1, :] = carry_new

    rank1 = jnp.sum(jnp.where(sel1, rank, 0.0), axis=1, keepdims=True)
    rank2 = jnp.sum(jnp.where(sel2, rank, 0.0), axis=1, keepdims=True)
    s1 = jnp.where(rank1 < CAP, i1 * CAP + rank1.astype(jnp.int32), DUMMY_SLOT)
    s2 = jnp.where(rank2 < CAP, i2 * CAP + rank2.astype(jnp.int32), DUMMY_SLOT)
    slots_ref[...] = jnp.where(lane == 0, s1, jnp.where(lane == 1, s2, DUMMY_SLOT))

    lf = jnp.minimum(carry_new, jnp.float32(CAP))
    sub8 = lax.broadcasted_iota(jnp.int32, (N_EXPERTS, LANES), 0)
    lane8 = lax.broadcasted_iota(jnp.int32, (N_EXPERTS, LANES), 1)
    lcol = jnp.sum(
        jnp.where(sub8 == lane8, jnp.broadcast_to(lf, (N_EXPERTS, LANES)), 0.0),
        axis=1, keepdims=True)
    cpos = lax.broadcasted_iota(jnp.int32, (N_EXPERTS, CAP), 1).astype(jnp.float32)
    smask_ref[...] = jnp.where(cpos < lcol, 1.0, 0.0)


N_BLOCKS_A = N_TOKENS // BLK_A
XPAD_ROWS = (N_BLOCKS_A + 1) * BLK_A


def _clamp_a(i):
    return (jnp.minimum(i, N_BLOCKS_A - 1), 0)


def _make_meta_call(interpret=False):
    return pl.pallas_call(
        _meta_body,
        interpret=interpret,
        grid=(N_BLOCKS_A + 1,),
        in_specs=[
            pl.BlockSpec((BLK_A, D_MODEL), _clamp_a),
            pl.BlockSpec((D_MODEL, LANES), lambda i: (0, 0)),
        ],
        out_specs=[
            pl.BlockSpec((BLK_A, LANES), _clamp_a),
            pl.BlockSpec((BLK_A, LANES), _clamp_a),
            pl.BlockSpec((BLK_A, LANES), _clamp_a),
            pl.BlockSpec((N_EXPERTS, CAP), lambda i: (0, 0)),
            pl.BlockSpec((BLK_A, D_MODEL), lambda i: (i, 0)),
        ],
        out_shape=[
            jax.ShapeDtypeStruct((N_TOKENS, LANES), jnp.float32),
            jax.ShapeDtypeStruct((N_TOKENS, LANES), jnp.float32),
            jax.ShapeDtypeStruct((N_TOKENS, LANES), jnp.int32),
            jax.ShapeDtypeStruct((N_EXPERTS, CAP), jnp.float32),
            jax.ShapeDtypeStruct((XPAD_ROWS, D_MODEL), jnp.float32),
        ],
        scratch_shapes=[pltpu.VMEM((8, LANES), jnp.float32)],
    )


_meta_call = _make_meta_call()


def _write_body(kept_ref, wk_ref, smask_ref, fw_ref, mk_ref):
    k8 = kept_ref[:, 0:N_EXPERTS]
    w8 = wk_ref[:, 0:N_EXPERTS]
    sm = smask_ref[...]
    mk_ref[...] = k8[:, :, None] * sm[None, :, :]
    fw_ref[...] = w8[:, :, None] * sm[None, :, :]


def _make_write_call(interpret=False):
    return pl.pallas_call(
        _write_body,
        interpret=interpret,
        grid=(N_TOKENS // BLK_B,),
        in_specs=[
            pl.BlockSpec((BLK_B, LANES), lambda i: (i, 0)),
            pl.BlockSpec((BLK_B, LANES), lambda i: (i, 0)),
            pl.BlockSpec((N_EXPERTS, CAP), lambda i: (0, 0)),
        ],
        out_specs=[
            pl.BlockSpec((BLK_B, N_EXPERTS, CAP), lambda i: (i, 0, 0)),
            pl.BlockSpec((BLK_B, N_EXPERTS, CAP), lambda i: (i, 0, 0)),
        ],
        out_shape=[
            jax.ShapeDtypeStruct((N_TOKENS, N_EXPERTS, CAP), jnp.float32),
            jax.ShapeDtypeStruct((N_TOKENS, N_EXPERTS, CAP), jnp.float32),
        ],
    )


_write_call = _make_write_call()


_SC_NC = 2
_SC_NS = 16
_SC_NW = _SC_NC * _SC_NS
_TBL = 5152
_PER_W = SLOTS // _SC_NW
_CHUNK = 80


def _gather_body(slots_hbm, xpad_hbm, out_hbm, tbl_v, slots_v, tok_v, tbl_sh,
                 myidx_v, rows_v, sem):
    cid = lax.axis_index("c")
    sid = lax.axis_index("s")
    wid = sid * _SC_NC + cid

    @pl.when(sid == 0)
    def _build():
        def _init(i, c):
            tbl_v[pl.ds(i * 16, 16)] = jnp.full((16,), PAD_ROW, jnp.int32)
            return c
        lax.fori_loop(0, _TBL // 16, _init, 0)
        pltpu.sync_copy(tbl_v, tbl_sh)
        pltpu.sync_copy(slots_hbm, slots_v)

        def _tok(i, c):
            tok_v[pl.ds(i * 16, 16)] = (i * 16 + lax.iota(jnp.int32, 16)) >> 1
            return c
        lax.fori_loop(0, (N_TOKENS * TOP_K) // 16, _tok, 0)
        pltpu.sync_copy(tok_v, tbl_sh.at[slots_v])

    plsc.subcore_barrier()
    base = wid * _PER_W
    pltpu.sync_copy(tbl_sh.at[pl.ds(base, _PER_W)], myidx_v)
    for ch in range(_PER_W // _CHUNK):
        cbase = ch * _CHUNK
        pltpu.async_copy(
            xpad_hbm.at[myidx_v.at[pl.ds(cbase, _CHUNK)]], rows_v, sem).wait()
        pltpu.sync_copy(rows_v, out_hbm.at[pl.ds(base + cbase, _CHUNK)])


@functools.cache
def _gather_call():
    return functools.partial(
        pl.kernel,
        mesh=plsc.VectorSubcoreMesh(core_axis_name="c", subcore_axis_name="s"),
        out_type=jax.ShapeDtypeStruct((SLOTS, D_MODEL), jnp.float32),
        scratch_types=[
            pltpu.VMEM((_TBL,), jnp.int32),
            pltpu.VMEM((N_TOKENS * TOP_K,), jnp.int32),
            pltpu.VMEM((N_TOKENS * TOP_K,), jnp.int32),
            pltpu.VMEM_SHARED((_TBL,), jnp.int32),
            pltpu.VMEM((_PER_W,), jnp.int32),
            pltpu.VMEM((_CHUNK, D_MODEL), jnp.float32),
            pltpu.SemaphoreType.DMA,
        ],
    )(_gather_body)


def kernel(x, W_gate):
    xf = x.reshape(N_TOKENS, D_MODEL)
    wt = jnp.zeros((D_MODEL, LANES), jnp.float32).at[:, :N_EXPERTS].set(W_gate.T)
    keptf, wk, slots128, smask, xpad = _meta_call(xf, wt)
    slots = slots128[:, :TOP_K].reshape(N_TOKENS * TOP_K)
    fw, mk = _write_call(keptf, wk, smask)
    eb = _gather_call()(slots, xpad)
    return fw, mk, eb.reshape(N_EXPERTS, CAP, D_MODEL)


# speedup vs baseline: 1.4467x; 1.0066x over previous
import functools
import math

import jax
import jax.numpy as jnp
from jax import lax
from jax.experimental import pallas as pl
from jax.experimental.pallas import tpu as pltpu
from jax.experimental.pallas import tpu_sc as plsc

D_MODEL = 1024
N_EXPERTS = 8
TOP_K = 2
N_TOKENS = 2048
CAP = math.floor(TOP_K * 1.25 * N_TOKENS / N_EXPERTS)
CAP += CAP % 2
SLOTS = N_EXPERTS * CAP
DUMMY_SLOT = SLOTS
PAD_ROW = N_TOKENS

LANES = 128
BLK_A = 256
BLK_B = 256


def _meta_body(x_ref, wt_ref, kept_ref, wk_ref, slots_ref, smask_ref,
               xpad_ref, carry_ref):
    i = pl.program_id(0)

    @pl.when(i == 0)
    def _():
        carry_ref[...] = jnp.zeros_like(carry_ref)

    @pl.when(i >= N_TOKENS // BLK_A)
    def _():
        xpad_ref[...] = jnp.zeros_like(xpad_ref)

    @pl.when(i < N_TOKENS // BLK_A)
    def _main():
        _meta_step(x_ref, wt_ref, kept_ref, wk_ref, slots_ref, smask_ref,
                   xpad_ref, carry_ref)


def _meta_step(x_ref, wt_ref, kept_ref, wk_ref, slots_ref, smask_ref,
               xpad_ref, carry_ref):
    xpad_ref[...] = x_ref[...]
    g = jnp.dot(x_ref[...], wt_ref[...], preferred_element_type=jnp.float32)
    lane = lax.broadcasted_iota(jnp.int32, (BLK_A, LANES), 1)
    neg = jnp.float32(-1e30)
    g = jnp.where(lane < N_EXPERTS, g, neg)

    m1 = jnp.max(g, axis=1, keepdims=True)
    i1 = jnp.min(jnp.where(g == m1, lane, LANES - 1), axis=1, keepdims=True)
    g2 = jnp.where(lane == i1, neg, g)
    m2 = jnp.max(g2, axis=1, keepdims=True)
    i2 = jnp.min(jnp.where(g2 == m2, lane, LANES - 1), axis=1, keepdims=True)

    t = jnp.exp(m2 - m1)
    denom = 1.0 + t
    ew1 = 1.0 / denom
    ew2 = t / denom
    sel1 = lane == i1
    sel2 = lane == i2
    self32 = jnp.where(sel1 | sel2, 1.0, 0.0).astype(jnp.float32)
    wte = jnp.where(sel1, ew1, 0.0) + jnp.where(sel2, ew2, 0.0)

    r_ = lax.broadcasted_iota(jnp.int32, (BLK_A, BLK_A), 0)
    c_ = lax.broadcasted_iota(jnp.int32, (BLK_A, BLK_A), 1)
    tril = jnp.where(r_ >= c_, 1.0, 0.0).astype(jnp.float32)
    cs = jnp.dot(tril, self32, preferred_element_type=jnp.float32)

    carry = carry_ref[0:1, :]
    rank = carry + cs - 1.0
    keptf = self32 * jnp.where(rank < CAP, 1.0, 0.0)
    kept_ref[...] = keptf
    wk_ref[...] = keptf * wte

    carry_new = carry + cs[BLK_A - 1:BLK_A, :]
    carry_ref[0:1, :] = carry_new

    rank1 = jnp.sum(jnp.where(sel1, rank, 0.0), axis=1, keepdims=True)
    rank2 = jnp.sum(jnp.where(sel2, rank, 0.0), axis=1, keepdims=True)
    s1 = jnp.where(rank1 < CAP, i1 * CAP + rank1.astype(jnp.int32), DUMMY_SLOT)
    s2 = jnp.where(rank2 < CAP, i2 * CAP + rank2.astype(jnp.int32), DUMMY_SLOT)
    slots_ref[...] = jnp.where(lane == 0, s1, jnp.where(lane == 1, s2, DUMMY_SLOT))

    lf = jnp.minimum(carry_new, jnp.float32(CAP))
    sub8 = lax.broadcasted_iota(jnp.int32, (N_EXPERTS, LANES), 0)
    lane8 = lax.broadcasted_iota(jnp.int32, (N_EXPERTS, LANES), 1)
    lcol = jnp.sum(
        jnp.where(sub8 == lane8, jnp.broadcast_to(lf, (N_EXPERTS, LANES)), 0.0),
        axis=1, keepdims=True)
    cpos = lax.broadcasted_iota(jnp.int32, (N_EXPERTS, CAP), 1).astype(jnp.float32)
    smask_ref[...] = jnp.where(cpos < lcol, 1.0, 0.0)


N_BLOCKS_A = N_TOKENS // BLK_A
XPAD_ROWS = (N_BLOCKS_A + 1) * BLK_A


def _clamp_a(i):
    return (jnp.minimum(i, N_BLOCKS_A - 1), 0)


def _make_meta_call(interpret=False):
    return pl.pallas_call(
        _meta_body,
        interpret=interpret,
        grid=(N_BLOCKS_A + 1,),
        in_specs=[
            pl.BlockSpec((BLK_A, D_MODEL), _clamp_a),
            pl.BlockSpec((D_MODEL, LANES), lambda i: (0, 0)),
        ],
        out_specs=[
            pl.BlockSpec((BLK_A, LANES), _clamp_a),
            pl.BlockSpec((BLK_A, LANES), _clamp_a),
            pl.BlockSpec((BLK_A, LANES), _clamp_a),
            pl.BlockSpec((N_EXPERTS, CAP), lambda i: (0, 0)),
            pl.BlockSpec((BLK_A, D_MODEL), lambda i: (i, 0)),
        ],
        out_shape=[
            jax.ShapeDtypeStruct((N_TOKENS, LANES), jnp.float32),
            jax.ShapeDtypeStruct((N_TOKENS, LANES), jnp.float32),
            jax.ShapeDtypeStruct((N_TOKENS, LANES), jnp.int32),
            jax.ShapeDtypeStruct((N_EXPERTS, CAP), jnp.float32),
            jax.ShapeDtypeStruct((XPAD_ROWS, D_MODEL), jnp.float32),
        ],
        scratch_shapes=[pltpu.VMEM((8, LANES), jnp.float32)],
    )


_meta_call = _make_meta_call()


def _write_body(kept_ref, wk_ref, smask_ref, fw_ref, mk_ref):
    k8 = kept_ref[:, 0:N_EXPERTS]
    w8 = wk_ref[:, 0:N_EXPERTS]
    sm = smask_ref[...]
    mk_ref[...] = k8[:, :, None] * sm[None, :, :]
    fw_ref[...] = w8[:, :, None] * sm[None, :, :]


def _make_write_call(interpret=False):
    return pl.pallas_call(
        _write_body,
        interpret=interpret,
        grid=(N_TOKENS // BLK_B,),
        in_specs=[
            pl.BlockSpec((BLK_B, LANES), lambda i: (i, 0)),
            pl.BlockSpec((BLK_B, LANES), lambda i: (i, 0)),
            pl.BlockSpec((N_EXPERTS, CAP), lambda i: (0, 0)),
        ],
        out_specs=[
            pl.BlockSpec((BLK_B, N_EXPERTS, CAP), lambda i: (i, 0, 0)),
            pl.BlockSpec((BLK_B, N_EXPERTS, CAP), lambda i: (i, 0, 0)),
        ],
        out_shape=[
            jax.ShapeDtypeStruct((N_TOKENS, N_EXPERTS, CAP), jnp.float32),
            jax.ShapeDtypeStruct((N_TOKENS, N_EXPERTS, CAP), jnp.float32),
        ],
    )


_write_call = _make_write_call()


_SC_NC = 2
_SC_NS = 16
_SC_NW = _SC_NC * _SC_NS
_TBL = 5152
_PER_W = SLOTS // _SC_NW
_CHUNK = 40
_INIT_PER = SLOTS // _SC_NS
_SCAT_PER = (N_TOKENS * TOP_K) // _SC_NS


def _gather_body(slots_hbm, xpad_hbm, out_hbm, pad_v, slots_v, tok_v, tbl_sh,
                 myidx_v, buf0, buf1, sem_g0, sem_g1, sem_w0, sem_w1):
    cid = lax.axis_index("c")
    sid = lax.axis_index("s")
    wid = sid * _SC_NC + cid

    def _pad(i, c):
        pad_v[pl.ds(i * 16, 16)] = jnp.full((16,), PAD_ROW, jnp.int32)
        return c
    lax.fori_loop(0, _INIT_PER // 16, _pad, 0)
    pltpu.sync_copy(pad_v, tbl_sh.at[pl.ds(sid * _INIT_PER, _INIT_PER)])

    pltpu.sync_copy(slots_hbm.at[pl.ds(sid * _SCAT_PER, _SCAT_PER)], slots_v)

    def _fix(i, c):
        s = slots_v[pl.ds(i * 16, 16)]
        slots_v[pl.ds(i * 16, 16)] = jnp.where(s >= SLOTS, SLOTS + sid, s)
        return c
    lax.fori_loop(0, _SCAT_PER // 16, _fix, 0)

    def _tok(i, c):
        tok_v[pl.ds(i * 16, 16)] = (
            sid * _SCAT_PER + i * 16 + lax.iota(jnp.int32, 16)) >> 1
        return c
    lax.fori_loop(0, _SCAT_PER // 16, _tok, 0)

    plsc.subcore_barrier()
    pltpu.sync_copy(tok_v, tbl_sh.at[slots_v])
    plsc.subcore_barrier()

    base = wid * _PER_W
    pltpu.sync_copy(tbl_sh.at[pl.ds(base, _PER_W)], myidx_v)

    def _gather(ch, buf, sem):
        return pltpu.async_copy(
            xpad_hbm.at[myidx_v.at[pl.ds(ch * _CHUNK, _CHUNK)]], buf, sem)

    def _write(ch, buf, sem):
        return pltpu.async_copy(
            buf, out_hbm.at[pl.ds(base + ch * _CHUNK, _CHUNK)], sem)

    g0 = _gather(0, buf0, sem_g0)
    g0.wait()
    w0 = _write(0, buf0, sem_w0)
    g1 = _gather(1, buf1, sem_g1)
    g1.wait()
    w0.wait()
    w1 = _write(1, buf1, sem_w1)
    g2 = _gather(2, buf0, sem_g0)
    g2.wait()
    w1.wait()
    w2 = _write(2, buf0, sem_w0)
    g3 = _gather(3, buf1, sem_g1)
    g3.wait()
    w2.wait()
    w3 = _write(3, buf1, sem_w1)
    w3.wait()


@functools.cache
def _gather_call():
    return functools.partial(
        pl.kernel,
        mesh=plsc.VectorSubcoreMesh(core_axis_name="c", subcore_axis_name="s"),
        out_type=jax.ShapeDtypeStruct((SLOTS, D_MODEL), jnp.float32),
        scratch_types=[
            pltpu.VMEM((_INIT_PER,), jnp.int32),
            pltpu.VMEM((_SCAT_PER,), jnp.int32),
            pltpu.VMEM((_SCAT_PER,), jnp.int32),
            pltpu.VMEM_SHARED((_TBL,), jnp.int32),
            pltpu.VMEM((_PER_W,), jnp.int32),
            pltpu.VMEM((_CHUNK, D_MODEL), jnp.float32),
            pltpu.VMEM((_CHUNK, D_MODEL), jnp.float32),
            pltpu.SemaphoreType.DMA,
            pltpu.SemaphoreType.DMA,
            pltpu.SemaphoreType.DMA,
            pltpu.SemaphoreType.DMA,
        ],
    )(_gather_body)


def kernel(x, W_gate):
    xf = x.reshape(N_TOKENS, D_MODEL)
    wt = jnp.zeros((D_MODEL, LANES), jnp.float32).at[:, :N_EXPERTS].set(W_gate.T)
    keptf, wk, slots128, smask, xpad = _meta_call(xf, wt)
    slots = slots128[:, :TOP_K].reshape(N_TOKENS * TOP_K)
    fw, mk = _write_call(keptf, wk, smask)
    eb = _gather_call()(slots, xpad)
    return fw, mk, eb.reshape(N_EXPERTS, CAP, D_MODEL)
